# baseline (speedup 1.0000x reference)
import jax
import jax.numpy as jnp
from jax.experimental import pallas as pl
from jax.experimental.pallas import tpu as pltpu

_BT = 1024
_NBUF = 8
_SIZES = [512, 512] + [1024] * 14 + [512, 512]
_TILES = []
_off = 0
for _s in _SIZES:
    _TILES.append((_off, _s))
    _off += _s
assert _off == 16384


def _make_body(m, k, n):
    def body(x_hbm, w_hbm, b_hbm, o_hbm, x_vmem, o_vmem, w_vmem, b_vmem,
             lsem, ssem, wsem):
        def load(i):
            b = i % _NBUF
            off, rows = _TILES[i]
            return pltpu.make_async_copy(
                x_hbm.at[pl.ds(off, rows), :],
                x_vmem.at[b, pl.ds(0, rows), :], lsem.at[b])

        def store(i):
            b = i % _NBUF
            off, rows = _TILES[i]
            return pltpu.make_async_copy(
                o_vmem.at[b, pl.ds(0, rows), :],
                o_hbm.at[pl.ds(off, rows), :], ssem.at[b])

        w_copy = pltpu.make_async_copy(w_hbm, w_vmem, wsem.at[0])
        b_copy = pltpu.make_async_copy(b_hbm, b_vmem, wsem.at[1])
        t_total = len(_TILES)
        load(0).start()
        w_copy.start()
        b_copy.start()
        for i in range(1, min(_NBUF, t_total)):
            load(i).start()
        w_copy.wait()
        b_copy.wait()
        for i in range(t_total):
            b = i % _NBUF
            rows = _TILES[i][1]
            load(i).wait()
            if i >= _NBUF:
                store(i - _NBUF).wait()
            acc = jax.lax.dot_general(
                x_vmem[b, :rows], w_vmem[:],
                dimension_numbers=(((1,), (1,)), ((), ())),
                preferred_element_type=jnp.float32,
            )
            o_vmem[b, :rows] = jnp.clip(acc + b_vmem[:], 0.0, 1.0)
            if i + _NBUF < t_total:
                load(i + _NBUF).start()
            store(i).start()
        for i in range(max(0, t_total - _NBUF), t_total):
            store(i).wait()

    return body


def kernel(x, weight, bias):
    m, k = x.shape
    n = weight.shape[0]
    bias2d = bias.reshape(1, n)
    return pl.pallas_call(
        _make_body(m, k, n),
        in_specs=[
            pl.BlockSpec(memory_space=pl.ANY),
            pl.BlockSpec(memory_space=pl.ANY),
            pl.BlockSpec(memory_space=pl.ANY),
        ],
        out_specs=pl.BlockSpec(memory_space=pl.ANY),
        out_shape=jax.ShapeDtypeStruct((m, n), jnp.float32),
        scratch_shapes=[
            pltpu.VMEM((_NBUF, _BT, k), jnp.float32),
            pltpu.VMEM((_NBUF, _BT, n), jnp.float32),
            pltpu.VMEM((n, k), jnp.float32),
            pltpu.VMEM((1, n), jnp.float32),
            pltpu.SemaphoreType.DMA((_NBUF,)),
            pltpu.SemaphoreType.DMA((_NBUF,)),
            pltpu.SemaphoreType.DMA((2,)),
        ],
    )(x, weight, bias2d)


# speedup vs baseline: 1.0969x; 1.0969x over previous
import jax
import jax.numpy as jnp
from jax.experimental import pallas as pl
from jax.experimental.pallas import tpu as pltpu

_BT = 1024
_NBUF = 6
_SIZES = [256, 256, 512] + [1024] * 14 + [512, 256, 256]
_TILES = []
_off = 0
for _s in _SIZES:
    _TILES.append((_off, _s))
    _off += _s
assert _off == 16384


def _make_body(m, k, n):
    def body(x_hbm, w_hbm, b_hbm, o_hbm, x_vmem, o_vmem, w_vmem, b_vmem,
             lsem, ssem, wsem):
        def load(i):
            b = i % _NBUF
            off, rows = _TILES[i]
            return pltpu.make_async_copy(
                x_hbm.at[pl.ds(off, rows), :],
                x_vmem.at[b, pl.ds(0, rows), :], lsem.at[b])

        def store(i):
            b = i % _NBUF
            off, rows = _TILES[i]
            return pltpu.make_async_copy(
                o_vmem.at[b, pl.ds(0, rows), :],
                o_hbm.at[pl.ds(off, rows), :], ssem.at[b])

        w_copy = pltpu.make_async_copy(w_hbm, w_vmem, wsem.at[0])
        b_copy = pltpu.make_async_copy(b_hbm, b_vmem, wsem.at[1])
        t_total = len(_TILES)
        load(0).start()
        w_copy.start()
        b_copy.start()
        for i in range(1, min(_NBUF, t_total)):
            load(i).start()
        w_copy.wait()
        b_copy.wait()
        for i in range(t_total):
            b = i % _NBUF
            rows = _TILES[i][1]
            load(i).wait()
            if i >= _NBUF:
                store(i - _NBUF).wait()
            acc = jax.lax.dot_general(
                x_vmem[b, :rows], w_vmem[:],
                dimension_numbers=(((1,), (1,)), ((), ())),
                preferred_element_type=jnp.float32,
            )
            o_vmem[b, :rows] = jnp.clip(acc + b_vmem[:], 0.0, 1.0)
            if i + _NBUF < t_total:
                load(i + _NBUF).start()
            store(i).start()
        for i in range(max(0, t_total - _NBUF), t_total):
            store(i).wait()

    return body


def kernel(x, weight, bias):
    m, k = x.shape
    n = weight.shape[0]
    bias2d = bias.reshape(1, n)
    return pl.pallas_call(
        _make_body(m, k, n),
        in_specs=[
            pl.BlockSpec(memory_space=pl.ANY),
            pl.BlockSpec(memory_space=pl.ANY),
            pl.BlockSpec(memory_space=pl.ANY),
        ],
        out_specs=pl.BlockSpec(memory_space=pl.ANY),
        out_shape=jax.ShapeDtypeStruct((m, n), jnp.float32),
        scratch_shapes=[
            pltpu.VMEM((_NBUF, _BT, k), jnp.float32),
            pltpu.VMEM((_NBUF, _BT, n), jnp.float32),
            pltpu.VMEM((n, k), jnp.float32),
            pltpu.VMEM((1, n), jnp.float32),
            pltpu.SemaphoreType.DMA((_NBUF,)),
            pltpu.SemaphoreType.DMA((_NBUF,)),
            pltpu.SemaphoreType.DMA((2,)),
        ],
    )(x, weight, bias2d)
